# baseline (speedup 1.0000x reference)
import functools

import jax
import jax.numpy as jnp
from jax import lax
from jax.experimental import pallas as pl
from jax.experimental.pallas import tpu as pltpu
from jax.experimental.pallas import tpu_sc as plsc

NC = 2
NS = 16
B = 128
CW = 16
NBUF = 6


def _round_up(a, m):
    return (a + m - 1) // m * m


def _dotT(a, w):
    return lax.dot_general(a, w, (((1,), (1,)), ((), ())),
                           preferred_element_type=jnp.float32)


def _split_cols(y):
    dh = y.shape[1] // 2
    return jnp.stack([y[:, :dh], y[:, dh:]], axis=0)


def _tc_in_proj(x, Wl, Wr):
    n, _ = x.shape
    do = Wl.shape[0]
    def body(x_ref, wl_ref, wr_ref, y_ref, z_ref):
        xx = x_ref[...]
        y_ref[...] = _split_cols(_dotT(xx, wl_ref[...]))
        z_ref[...] = _dotT(xx, wr_ref[...])
    return pl.pallas_call(
        body,
        out_shape=[jax.ShapeDtypeStruct((2, n, do // 2), jnp.float32),
                   jax.ShapeDtypeStruct((n, Wr.shape[0]), jnp.float32)],
    )(x, Wl, Wr)


def _tc_mid(psum, cnt, z1, bl1, Wl2, Wr2, n):
    do = Wl2.shape[0]
    def body(p_ref, c_ref, z_ref, b_ref, wl_ref, wr_ref, y2_ref, z2_ref):
        p = p_ref[...]
        s = jnp.concatenate([p[0, :n, :], p[1, :n, :]], axis=1)
        cn = c_ref[...]
        cc = jnp.maximum(cn[0, :n] + cn[1, :n], 1.0)
        h = jnp.maximum(s / cc[:, None] + b_ref[...] + z_ref[...], 0.0)
        y2_ref[...] = _split_cols(_dotT(h, wl_ref[...]))
        z2_ref[...] = _dotT(h, wr_ref[...])
    return pl.pallas_call(
        body,
        out_shape=[jax.ShapeDtypeStruct((2, n, do // 2), jnp.float32),
                   jax.ShapeDtypeStruct((n, Wr2.shape[0]), jnp.float32)],
    )(psum, cnt, z1, bl1, Wl2, Wr2)


def _tc_final(psum, cnt, z2, bl2, n):
    d = z2.shape[1]
    def body(p_ref, c_ref, z_ref, b_ref, o_ref):
        p = p_ref[...]
        s = jnp.concatenate([p[0, :n, :], p[1, :n, :]], axis=1)
        cn = c_ref[...]
        cc = jnp.maximum(cn[0, :n] + cn[1, :n], 1.0)
        o_ref[...] = s / cc[:, None] + b_ref[...] + z_ref[...]
    return pl.pallas_call(
        body,
        out_shape=jax.ShapeDtypeStruct((n, d), jnp.float32),
    )(psum, cnt, z2, bl2)


@functools.lru_cache(maxsize=None)
def _make_sc_segsum(dh, npad, rpt, cpw, with_cnt):
    mesh = plsc.VectorSubcoreMesh(core_axis_name="c", subcore_axis_name="s")
    out_type = [jax.ShapeDtypeStruct((NC, npad, dh), jnp.float32)]
    scratch = (
        [pltpu.VMEM((B, dh), jnp.float32) for _ in range(NBUF)]
        + [pltpu.VMEM((2, B), jnp.int32) for _ in range(NBUF)]
        + [pltpu.SemaphoreType.DMA for _ in range(2 * NBUF)]
    )
    if with_cnt:
        out_type.append(jax.ShapeDtypeStruct((NC, npad, CW), jnp.float32))
        scratch += [
            pltpu.VMEM((B, CW), jnp.float32),
            pltpu.VMEM((rpt, CW), jnp.float32),
            pltpu.VMEM_SHARED((npad, CW), jnp.float32),
        ]
    scratch.append(pltpu.VMEM_SHARED((npad, dh), jnp.float32))

    half = cpw // 2

    def body(y_hbm, idx_hbm, zr_hbm, ones_hbm, zc_hbm, *rest):
        if with_cnt:
            psum_hbm, cnt_hbm = rest[0], rest[1]
            rest = rest[2:]
        else:
            psum_hbm = rest[0]
            rest = rest[1:]
        bufs = rest[0:NBUF]
        ibs = rest[NBUF:2 * NBUF]
        gsems = rest[2 * NBUF:3 * NBUF]
        isems = rest[3 * NBUF:4 * NBUF]
        rest = rest[4 * NBUF:]
        if with_cnt:
            onesb, zcb, cacc = rest[0], rest[1], rest[2]
            rest = rest[3:]
        acc = rest[0]

        c = lax.axis_index("c")
        s = lax.axis_index("s")
        r0 = s * rpt

        pltpu.sync_copy(zr_hbm, bufs[0])
        for q in range(rpt // B):
            pltpu.sync_copy(bufs[0], acc.at[pl.ds(r0 + q * B, B)])
        if with_cnt:
            pltpu.sync_copy(ones_hbm, onesb)
            pltpu.sync_copy(zc_hbm, zcb)
            pltpu.sync_copy(zcb, cacc.at[pl.ds(r0, rpt)])
        plsc.subcore_barrier()

        def start_idx(k, j):
            pltpu.async_copy(idx_hbm.at[s, k], ibs[j], isems[j])

        def wait_idx(j):
            pltpu.make_async_copy(idx_hbm.at[s, 0], ibs[j], isems[j]).wait()

        def start_gather(k, j):
            pltpu.async_copy(y_hbm.at[c].at[ibs[j].at[0]], bufs[j], gsems[j])

        def wait_gather(j):
            pltpu.make_async_copy(zr_hbm, bufs[j], gsems[j]).wait()

        def scatter(k, j):
            pltpu.sync_copy(bufs[j], acc.at[ibs[j].at[1]], add=True)
            if with_cnt:
                do_cnt = ((c == 0) & (k < half)) | ((c != 0) & (k >= half))
                @pl.when(do_cnt)
                def _():
                    pltpu.sync_copy(onesb, cacc.at[ibs[j].at[1]], add=True)

        for j in range(NBUF):
            start_idx(j, j)
        for j in range(4):
            wait_idx(j)
            start_gather(j, j)

        @pl.loop(0, (cpw - NBUF) // NBUF)
        def _(gi):
            for j in range(NBUF):
                k = NBUF * gi + j
                wait_gather(j)
                scatter(k, j)
                start_idx(k + NBUF, j)
                wait_idx((j + 4) % NBUF)
                start_gather(k + 4, (j + 4) % NBUF)

        for j in range(NBUF):
            k = cpw - NBUF + j
            wait_gather(j)
            if j < 2:
                wait_idx((j + 4) % NBUF)
                start_gather(k + 4, (j + 4) % NBUF)
            scatter(k, j)

        plsc.subcore_barrier()

        for q in range(rpt // B):
            r = r0 + q * B
            pltpu.sync_copy(acc.at[pl.ds(r, B)], bufs[0])
            pltpu.sync_copy(bufs[0], psum_hbm.at[c, pl.ds(r, B)])
        if with_cnt:
            pltpu.sync_copy(cacc.at[pl.ds(r0, rpt)], zcb)
            pltpu.sync_copy(zcb, cnt_hbm.at[c, pl.ds(r0, rpt)])

    return pl.kernel(body, out_type=out_type, mesh=mesh,
                     scratch_types=scratch,
                     compiler_params=pltpu.CompilerParams(
                         use_tc_tiling_on_sc=False))


def kernel(x, edge_index, Wl1, bl1, Wr1, Wl2, bl2, Wr2):
    n, d = x.shape
    e = edge_index.shape[1]
    dh = d // 2

    cpw = _round_up(-(-e // (NS * B)), 2 * NBUF)
    e_pad = NS * cpw * B
    rpt = _round_up(-(-(n + 1) // NS), B)
    npad = NS * rpt

    pad = e_pad - e
    src_p = jnp.concatenate([edge_index[0], jnp.zeros((pad,), jnp.int32)]
                            ).reshape(NS, cpw, B)
    dst_p = jnp.concatenate([edge_index[1], jnp.full((pad,), n, jnp.int32)]
                            ).reshape(NS, cpw, B)
    idx = jnp.stack([src_p, dst_p], axis=2)
    zr = jnp.zeros((B, dh), jnp.float32)
    onesc = jnp.ones((B, CW), jnp.float32)
    zc = jnp.zeros((rpt, CW), jnp.float32)

    bl1r = bl1.reshape(1, -1)
    bl2r = bl2.reshape(1, -1)

    y1, z1 = _tc_in_proj(x, Wl1, Wr1)
    psum1, cnt = _make_sc_segsum(dh, npad, rpt, cpw, True)(
        y1, idx, zr, onesc, zc)
    cnt2 = cnt[:, :, 0]
    y2, z2 = _tc_mid(psum1, cnt2, z1, bl1r, Wl2, Wr2, n)
    dh2 = Wl2.shape[0] // 2
    (psum2,) = _make_sc_segsum(dh2, npad, rpt, cpw, False)(
        y2, idx, zr, onesc, zc)
    return _tc_final(psum2, cnt2, z2, bl2r, n)


# speedup vs baseline: 2.3156x; 2.3156x over previous
import functools

import jax
import jax.numpy as jnp
from jax import lax
from jax.experimental import pallas as pl
from jax.experimental.pallas import tpu as pltpu
from jax.experimental.pallas import tpu_sc as plsc

NC = 2
NS = 16
B = 128
CW = 16
NBUF = 4


def _round_up(a, m):
    return (a + m - 1) // m * m


def _dotT(a, w):
    return lax.dot_general(a, w, (((1,), (1,)), ((), ())),
                           preferred_element_type=jnp.float32)


def _split_cols(y):
    dh = y.shape[1] // 2
    return jnp.stack([y[:, :dh], y[:, dh:]], axis=0)


def _tc_in_proj(x, Wl, Wr):
    n, _ = x.shape
    do = Wl.shape[0]
    def body(x_ref, wl_ref, wr_ref, y_ref, z_ref):
        xx = x_ref[...]
        y_ref[...] = _split_cols(_dotT(xx, wl_ref[...]))
        z_ref[...] = _dotT(xx, wr_ref[...])
    return pl.pallas_call(
        body,
        out_shape=[jax.ShapeDtypeStruct((2, n, do // 2), jnp.float32),
                   jax.ShapeDtypeStruct((n, Wr.shape[0]), jnp.float32)],
    )(x, Wl, Wr)


def _tc_mid(psum, cnt, z1, bl1, Wl2, Wr2, n):
    do = Wl2.shape[0]
    def body(p_ref, c_ref, z_ref, b_ref, wl_ref, wr_ref, y2_ref, z2_ref):
        p = p_ref[...]
        s = jnp.concatenate([p[0, :n, :], p[1, :n, :]], axis=1)
        cn = c_ref[...]
        cc = jnp.maximum(cn[0, :n] + cn[1, :n], 1.0)
        h = jnp.maximum(s / cc[:, None] + b_ref[...] + z_ref[...], 0.0)
        y2_ref[...] = _split_cols(_dotT(h, wl_ref[...]))
        z2_ref[...] = _dotT(h, wr_ref[...])
    return pl.pallas_call(
        body,
        out_shape=[jax.ShapeDtypeStruct((2, n, do // 2), jnp.float32),
                   jax.ShapeDtypeStruct((n, Wr2.shape[0]), jnp.float32)],
    )(psum, cnt, z1, bl1, Wl2, Wr2)


def _tc_final(psum, cnt, z2, bl2, n):
    d = z2.shape[1]
    def body(p_ref, c_ref, z_ref, b_ref, o_ref):
        p = p_ref[...]
        s = jnp.concatenate([p[0, :n, :], p[1, :n, :]], axis=1)
        cn = c_ref[...]
        cc = jnp.maximum(cn[0, :n] + cn[1, :n], 1.0)
        o_ref[...] = s / cc[:, None] + b_ref[...] + z_ref[...]
    return pl.pallas_call(
        body,
        out_shape=jax.ShapeDtypeStruct((n, d), jnp.float32),
    )(psum, cnt, z2, bl2)


@functools.lru_cache(maxsize=None)
def _make_sc_segsum(dh, npad, rpt, cpw, with_cnt):
    mesh = plsc.VectorSubcoreMesh(core_axis_name="c", subcore_axis_name="s")
    out_type = [jax.ShapeDtypeStruct((NC, npad, dh), jnp.float32)]
    scratch = (
        [pltpu.VMEM((B, dh), jnp.float32) for _ in range(NBUF)]
        + [pltpu.VMEM((2, B), jnp.int32) for _ in range(NBUF)]
        + [pltpu.SemaphoreType.DMA for _ in range(2 * NBUF)]
    )
    if with_cnt:
        out_type.append(jax.ShapeDtypeStruct((NC, npad, CW), jnp.float32))
        scratch += [
            pltpu.VMEM((B, CW), jnp.float32),
            pltpu.VMEM((rpt, CW), jnp.float32),
            pltpu.VMEM_SHARED((npad, CW), jnp.float32),
        ]
    scratch.append(pltpu.VMEM_SHARED((npad, dh), jnp.float32))

    half = cpw // 2

    def body(y_hbm, idx_hbm, zr_hbm, ones_hbm, zc_hbm, *rest):
        if with_cnt:
            psum_hbm, cnt_hbm = rest[0], rest[1]
            rest = rest[2:]
        else:
            psum_hbm = rest[0]
            rest = rest[1:]
        bufs = rest[0:NBUF]
        ibs = rest[NBUF:2 * NBUF]
        gsems = rest[2 * NBUF:3 * NBUF]
        isems = rest[3 * NBUF:4 * NBUF]
        rest = rest[4 * NBUF:]
        if with_cnt:
            onesb, zcb, cacc = rest[0], rest[1], rest[2]
            rest = rest[3:]
        acc = rest[0]

        c = lax.axis_index("c")
        s = lax.axis_index("s")
        r0 = s * rpt

        pltpu.sync_copy(zr_hbm, bufs[0])
        for q in range(rpt // B):
            pltpu.sync_copy(bufs[0], acc.at[pl.ds(r0 + q * B, B)])
        if with_cnt:
            pltpu.sync_copy(ones_hbm, onesb)
            pltpu.sync_copy(zc_hbm, zcb)
            pltpu.sync_copy(zcb, cacc.at[pl.ds(r0, rpt)])
        plsc.subcore_barrier()

        def start_idx(k, j):
            pltpu.async_copy(idx_hbm.at[s, k], ibs[j], isems[j])

        def wait_idx(j):
            pltpu.make_async_copy(idx_hbm.at[s, 0], ibs[j], isems[j]).wait()

        def start_gather(k, j):
            pltpu.async_copy(y_hbm.at[c].at[ibs[j].at[0]], bufs[j], gsems[j])

        def wait_gather(j):
            pltpu.make_async_copy(zr_hbm, bufs[j], gsems[j]).wait()

        def scatter(k, j):
            pltpu.sync_copy(bufs[j], acc.at[ibs[j].at[1]], add=True)
            if with_cnt:
                do_cnt = ((c == 0) & (k < half)) | ((c != 0) & (k >= half))
                @pl.when(do_cnt)
                def _():
                    pltpu.sync_copy(onesb, cacc.at[ibs[j].at[1]], add=True)

        for j in range(NBUF):
            start_idx(j, j)
        for j in range(3):
            wait_idx(j)
            start_gather(j, j)

        @pl.loop(0, (cpw - NBUF) // NBUF)
        def _(gi):
            for j in range(NBUF):
                k = NBUF * gi + j
                wait_gather(j)
                scatter(k, j)
                start_idx(k + NBUF, j)
                wait_idx((j + 3) % NBUF)
                start_gather(k + 3, (j + 3) % NBUF)

        for j in range(NBUF):
            k = cpw - NBUF + j
            wait_gather(j)
            if j < 1:
                wait_idx((j + 3) % NBUF)
                start_gather(k + 3, (j + 3) % NBUF)
            scatter(k, j)

        plsc.subcore_barrier()

        for q in range(rpt // B):
            r = r0 + q * B
            pltpu.sync_copy(acc.at[pl.ds(r, B)], bufs[0])
            pltpu.sync_copy(bufs[0], psum_hbm.at[c, pl.ds(r, B)])
        if with_cnt:
            pltpu.sync_copy(cacc.at[pl.ds(r0, rpt)], zcb)
            pltpu.sync_copy(zcb, cnt_hbm.at[c, pl.ds(r0, rpt)])

    return pl.kernel(body, out_type=out_type, mesh=mesh,
                     scratch_types=scratch,
                     compiler_params=pltpu.CompilerParams(
                         use_tc_tiling_on_sc=False))


def kernel(x, edge_index, Wl1, bl1, Wr1, Wl2, bl2, Wr2):
    n, d = x.shape
    e = edge_index.shape[1]
    dh = d // 2

    cpw = _round_up(-(-e // (NS * B)), 2 * NBUF)
    e_pad = NS * cpw * B
    rpt = _round_up(-(-(n + 1) // NS), B)
    npad = NS * rpt

    pad = e_pad - e
    src_p = jnp.concatenate([edge_index[0], jnp.zeros((pad,), jnp.int32)]
                            ).reshape(NS, cpw, B)
    dst_p = jnp.concatenate([edge_index[1], jnp.full((pad,), n, jnp.int32)]
                            ).reshape(NS, cpw, B)
    idx = jnp.stack([src_p, dst_p], axis=2)
    zr = jnp.zeros((B, dh), jnp.float32)
    onesc = jnp.ones((B, CW), jnp.float32)
    zc = jnp.zeros((rpt, CW), jnp.float32)

    bl1r = bl1.reshape(1, -1)
    bl2r = bl2.reshape(1, -1)

    y1, z1 = _tc_in_proj(x, Wl1, Wr1)
    psum1, cnt = _make_sc_segsum(dh, npad, rpt, cpw, True)(
        y1, idx, zr, onesc, zc)
    cnt2 = cnt[:, :, 0]
    y2, z2 = _tc_mid(psum1, cnt2, z1, bl1r, Wl2, Wr2, n)
    dh2 = Wl2.shape[0] // 2
    (psum2,) = _make_sc_segsum(dh2, npad, rpt, cpw, False)(
        y2, idx, zr, onesc, zc)
    return _tc_final(psum2, cnt2, z2, bl2r, n)


# speedup vs baseline: 3.8657x; 1.6694x over previous
import functools

import jax
import jax.numpy as jnp
from jax import lax
from jax.experimental import pallas as pl
from jax.experimental.pallas import tpu as pltpu
from jax.experimental.pallas import tpu_sc as plsc

NC = 2
NS = 16
B = 128
CW = 16
NBUF = 3


def _round_up(a, m):
    return (a + m - 1) // m * m


def _dotT(a, w):
    return lax.dot_general(a, w, (((1,), (1,)), ((), ())),
                           preferred_element_type=jnp.float32)


def _split_cols(y):
    dh = y.shape[1] // 2
    return jnp.stack([y[:, :dh], y[:, dh:]], axis=0)


def _tc_in_proj(x, Wl, Wr):
    n, _ = x.shape
    do = Wl.shape[0]
    def body(x_ref, wl_ref, wr_ref, y_ref, z_ref):
        xx = x_ref[...]
        y_ref[...] = _split_cols(_dotT(xx, wl_ref[...]))
        z_ref[...] = _dotT(xx, wr_ref[...])
    return pl.pallas_call(
        body,
        out_shape=[jax.ShapeDtypeStruct((2, n, do // 2), jnp.float32),
                   jax.ShapeDtypeStruct((n, Wr.shape[0]), jnp.float32)],
    )(x, Wl, Wr)


def _tc_mid(psum, cnt, z1, bl1, Wl2, Wr2, n):
    do = Wl2.shape[0]
    def body(p_ref, c_ref, z_ref, b_ref, wl_ref, wr_ref, y2_ref, z2_ref):
        p = p_ref[...]
        s = jnp.concatenate([p[0, :n, :], p[1, :n, :]], axis=1)
        cn = c_ref[...]
        cc = jnp.maximum(cn[0, :n] + cn[1, :n], 1.0)
        h = jnp.maximum(s / cc[:, None] + b_ref[...] + z_ref[...], 0.0)
        y2_ref[...] = _split_cols(_dotT(h, wl_ref[...]))
        z2_ref[...] = _dotT(h, wr_ref[...])
    return pl.pallas_call(
        body,
        out_shape=[jax.ShapeDtypeStruct((2, n, do // 2), jnp.float32),
                   jax.ShapeDtypeStruct((n, Wr2.shape[0]), jnp.float32)],
    )(psum, cnt, z1, bl1, Wl2, Wr2)


def _tc_final(psum, cnt, z2, bl2, n):
    d = z2.shape[1]
    def body(p_ref, c_ref, z_ref, b_ref, o_ref):
        p = p_ref[...]
        s = jnp.concatenate([p[0, :n, :], p[1, :n, :]], axis=1)
        cn = c_ref[...]
        cc = jnp.maximum(cn[0, :n] + cn[1, :n], 1.0)
        o_ref[...] = s / cc[:, None] + b_ref[...] + z_ref[...]
    return pl.pallas_call(
        body,
        out_shape=jax.ShapeDtypeStruct((n, d), jnp.float32),
    )(psum, cnt, z2, bl2)


@functools.lru_cache(maxsize=None)
def _make_sc_segsum(n, dh, npad, rpt, cpw, with_cnt):
    mesh = plsc.VectorSubcoreMesh(core_axis_name="c", subcore_axis_name="s")
    out_type = [jax.ShapeDtypeStruct((NC, npad, dh), jnp.float32)]
    scratch = (
        [pltpu.VMEM((B, dh), jnp.float32) for _ in range(NBUF)]
        + [pltpu.VMEM((2, B), jnp.int32) for _ in range(NBUF)]
        + [pltpu.SemaphoreType.DMA for _ in range(2 * NBUF)]
    )
    if with_cnt:
        out_type.append(jax.ShapeDtypeStruct((NC, npad, CW), jnp.float32))
        scratch += [
            pltpu.VMEM((B, CW), jnp.float32),
            pltpu.VMEM((B, CW), jnp.float32),
            pltpu.VMEM_SHARED((npad, CW), jnp.float32),
        ]
    scratch.append(pltpu.VMEM_SHARED((npad, dh), jnp.float32))
    scratch.append(pltpu.VMEM_SHARED((n, dh), jnp.float32))

    half = cpw // 2

    def body(y_hbm, idx_hbm, zr_hbm, ones_hbm, zc_hbm, *rest):
        if with_cnt:
            psum_hbm, cnt_hbm = rest[0], rest[1]
            rest = rest[2:]
        else:
            psum_hbm = rest[0]
            rest = rest[1:]
        bufs = rest[0:NBUF]
        ibs = rest[NBUF:2 * NBUF]
        gsems = rest[2 * NBUF:3 * NBUF]
        isems = rest[3 * NBUF:4 * NBUF]
        rest = rest[4 * NBUF:]
        if with_cnt:
            onesb, zcb, cacc = rest[0], rest[1], rest[2]
            rest = rest[3:]
        acc = rest[0]
        yspm = rest[1]

        c = lax.axis_index("c")
        s = lax.axis_index("s")
        r0 = s * rpt

        wrows = n // NS
        t0 = s * wrows
        pltpu.sync_copy(y_hbm.at[c, pl.ds(t0, wrows)],
                        yspm.at[pl.ds(t0, wrows)])
        pltpu.sync_copy(zr_hbm, bufs[0])
        for q in range(rpt // B):
            pltpu.sync_copy(bufs[0], acc.at[pl.ds(r0 + q * B, B)])
        if with_cnt:
            pltpu.sync_copy(ones_hbm, onesb)
            pltpu.sync_copy(zc_hbm, zcb)
            for q in range(rpt // B):
                pltpu.sync_copy(zcb, cacc.at[pl.ds(r0 + q * B, B)])
        plsc.subcore_barrier()

        def start_idx(k, j):
            pltpu.async_copy(idx_hbm.at[s, k], ibs[j], isems[j])

        def wait_idx(j):
            pltpu.make_async_copy(idx_hbm.at[s, 0], ibs[j], isems[j]).wait()

        def start_gather(k, j):
            pltpu.async_copy(yspm.at[ibs[j].at[0]], bufs[j], gsems[j])

        def wait_gather(j):
            pltpu.make_async_copy(zr_hbm, bufs[j], gsems[j]).wait()

        def scatter(k, j):
            pltpu.sync_copy(bufs[j], acc.at[ibs[j].at[1]], add=True)
            if with_cnt:
                do_cnt = ((c == 0) & (k < half)) | ((c != 0) & (k >= half))
                @pl.when(do_cnt)
                def _():
                    pltpu.sync_copy(onesb, cacc.at[ibs[j].at[1]], add=True)

        for j in range(NBUF):
            start_idx(j, j)
        for j in range(2):
            wait_idx(j)
            start_gather(j, j)

        @pl.loop(0, (cpw - NBUF) // NBUF)
        def _(gi):
            for j in range(NBUF):
                k = NBUF * gi + j
                wait_gather(j)
                scatter(k, j)
                start_idx(k + NBUF, j)
                wait_idx((j + 2) % NBUF)
                start_gather(k + 2, (j + 2) % NBUF)

        for j in range(NBUF):
            k = cpw - NBUF + j
            wait_gather(j)
            if j < 1:
                wait_idx((j + 2) % NBUF)
                start_gather(k + 2, (j + 2) % NBUF)
            scatter(k, j)

        plsc.subcore_barrier()

        for q in range(rpt // B):
            r = r0 + q * B
            pltpu.sync_copy(acc.at[pl.ds(r, B)], bufs[0])
            pltpu.sync_copy(bufs[0], psum_hbm.at[c, pl.ds(r, B)])
        if with_cnt:
            for q in range(rpt // B):
                r = r0 + q * B
                pltpu.sync_copy(cacc.at[pl.ds(r, B)], zcb)
                pltpu.sync_copy(zcb, cnt_hbm.at[c, pl.ds(r, B)])

    return pl.kernel(body, out_type=out_type, mesh=mesh,
                     scratch_types=scratch,
                     compiler_params=pltpu.CompilerParams(
                         use_tc_tiling_on_sc=False))


def kernel(x, edge_index, Wl1, bl1, Wr1, Wl2, bl2, Wr2):
    n, d = x.shape
    e = edge_index.shape[1]
    dh = d // 2

    cpw = _round_up(-(-e // (NS * B)), 2 * NBUF)
    e_pad = NS * cpw * B
    rpt = _round_up(-(-(n + 1) // NS), B)
    npad = NS * rpt

    pad = e_pad - e
    src_p = jnp.concatenate([edge_index[0], jnp.zeros((pad,), jnp.int32)]
                            ).reshape(NS, cpw, B)
    dst_p = jnp.concatenate([edge_index[1], jnp.full((pad,), n, jnp.int32)]
                            ).reshape(NS, cpw, B)
    idx = jnp.stack([src_p, dst_p], axis=2)
    zr = jnp.zeros((B, dh), jnp.float32)
    onesc = jnp.ones((B, CW), jnp.float32)
    zc = jnp.zeros((B, CW), jnp.float32)

    bl1r = bl1.reshape(1, -1)
    bl2r = bl2.reshape(1, -1)

    y1, z1 = _tc_in_proj(x, Wl1, Wr1)
    psum1, cnt = _make_sc_segsum(n, dh, npad, rpt, cpw, True)(
        y1, idx, zr, onesc, zc)
    cnt2 = cnt[:, :, 0]
    y2, z2 = _tc_mid(psum1, cnt2, z1, bl1r, Wl2, Wr2, n)
    dh2 = Wl2.shape[0] // 2
    (psum2,) = _make_sc_segsum(n, dh2, npad, rpt, cpw, False)(
        y2, idx, zr, onesc, zc)
    return _tc_final(psum2, cnt2, z2, bl2r, n)
